# baseline (speedup 1.0000x reference)
import functools

import jax
import jax.numpy as jnp
from jax import lax
from jax.experimental import pallas as pl
from jax.experimental.pallas import tpu as pltpu
from jax.experimental.pallas import tpu_sc as plsc

NC = 2
NS = 16
NW = NC * NS
LANES = 16
CHUNK = 128
MASK16 = 0xFFFF


def _mesh():
    return plsc.VectorSubcoreMesh(core_axis_name="c", subcore_axis_name="s")


def _unpack(packed_all, j, si_c, di_c, cg=CHUNK):
    @pl.loop(0, cg // LANES)
    def _(k):
        p = packed_all[pl.ds(j * cg + k * LANES, LANES)]
        si_c[pl.ds(k * LANES, LANES)] = jnp.bitwise_and(p, MASK16)
        di_c[pl.ds(k * LANES, LANES)] = jnp.right_shift(p, 16)


def _make_sc_degree(E, N_pad):
    epw = E // NW
    n_full = epw // CHUNK
    tail = epw - n_full * CHUNK
    assert E % NW == 0 and epw % 8 == 0 and tail % LANES == 0
    rps = N_pad // NS
    assert rps % LANES == 0
    FIRE = 6
    assert n_full % FIRE == 0

    @functools.partial(
        pl.kernel,
        out_type=jax.ShapeDtypeStruct((NC, N_pad), jnp.float32),
        mesh=_mesh(),
        scratch_types=[
            pltpu.VMEM((epw,), jnp.int32),
            [pltpu.VMEM((CHUNK,), jnp.int32) for _ in range(FIRE)],
            pltpu.VMEM((CHUNK,), jnp.int32),
            pltpu.VMEM((CHUNK,), jnp.float32),
            pltpu.VMEM((rps,), jnp.float32),
            pltpu.VMEM_SHARED((N_pad,), jnp.float32),
            pltpu.SemaphoreType.DMA,
        ],
    )
    def deg_kernel(pk_hbm, out_hbm, pk_all, di_cs, di_scr, ones_v, zbuf_v,
                   acc_sh, sem):
        c = lax.axis_index("c")
        s = lax.axis_index("s")
        wid = c * NS + s

        @pl.loop(0, CHUNK // LANES)
        def _(i):
            ones_v[pl.ds(i * LANES, LANES)] = jnp.ones((LANES,), jnp.float32)

        @pl.loop(0, rps // LANES)
        def _(i):
            zbuf_v[pl.ds(i * LANES, LANES)] = jnp.zeros((LANES,), jnp.float32)

        pltpu.sync_copy(zbuf_v, acc_sh.at[pl.ds(s * rps, rps)])
        pltpu.sync_copy(pk_hbm.at[pl.ds(wid * epw, epw)], pk_all)
        plsc.subcore_barrier()

        @pl.loop(0, n_full // FIRE)
        def _(i):
            for b in range(FIRE):
                _unpack(pk_all, i * FIRE + b, di_scr, di_cs[b])
                pltpu.async_copy(ones_v, acc_sh.at[di_cs[b]], sem, add=True)
            for b in range(FIRE):
                pltpu.make_async_copy(ones_v, acc_sh.at[di_cs[0]], sem).wait()

        if tail:
            @pl.loop(0, tail // LANES)
            def _(k):
                p = pk_all[pl.ds(n_full * CHUNK + k * LANES, LANES)]
                di_cs[0][pl.ds(k * LANES, LANES)] = jnp.right_shift(p, 16)
            pltpu.sync_copy(ones_v.at[pl.ds(0, tail)],
                            acc_sh.at[di_cs[0].at[pl.ds(0, tail)]], add=True)

        plsc.subcore_barrier()
        pltpu.sync_copy(acc_sh.at[pl.ds(s * rps, rps)],
                        out_hbm.at[c, pl.ds(s * rps, rps)])

    return deg_kernel


def _make_sc_agg(E, N_pad, D, tc_tiling=True, cg=CHUNK, nb=2):
    epw = E // NW
    n_full = epw // cg
    tail = epw - n_full * cg
    assert E % NW == 0 and epw % 8 == 0 and tail % LANES == 0
    assert n_full % nb == 0
    rps = N_pad // NS
    t_sz = tail if tail else LANES

    @functools.partial(
        pl.kernel,
        out_type=jax.ShapeDtypeStruct((NC, N_pad, D), jnp.float32),
        mesh=_mesh(),
        scratch_types=[
            pltpu.VMEM((epw,), jnp.int32),
            [pltpu.VMEM((cg,), jnp.int32) for _ in range(nb)],
            [pltpu.VMEM((cg,), jnp.int32) for _ in range(nb)],
            [pltpu.VMEM((cg, D), jnp.float32) for _ in range(nb)],
            pltpu.VMEM((t_sz,), jnp.int32),
            pltpu.VMEM((t_sz,), jnp.int32),
            pltpu.VMEM((t_sz, D), jnp.float32),
            pltpu.VMEM_SHARED((N_pad, D), jnp.float32),
            [pltpu.SemaphoreType.DMA for _ in range(nb)],
        ],
        compiler_params=pltpu.CompilerParams(use_tc_tiling_on_sc=tc_tiling),
    )
    def agg_kernel(h_hbm, pk_hbm, zero_hbm, out_hbm,
                   pk_all, si_c, di_c, rows, si_t, di_t, rows_t, acc_sh, g):
        c = lax.axis_index("c")
        s = lax.axis_index("s")
        wid = c * NS + s

        pltpu.sync_copy(zero_hbm.at[pl.ds(s * rps, rps)],
                        acc_sh.at[pl.ds(s * rps, rps)])
        pltpu.sync_copy(pk_hbm.at[pl.ds(wid * epw, epw)], pk_all)
        plsc.subcore_barrier()

        for b in range(nb):
            _unpack(pk_all, b, si_c[b], di_c[b], cg)
            pltpu.async_copy(h_hbm.at[si_c[b]], rows[b], g[b])

        @pl.loop(0, n_full // nb)
        def _(i):
            j = i * nb
            for b in range(nb):
                pltpu.make_async_copy(h_hbm.at[si_c[b]], rows[b],
                                      g[b]).wait()
                pltpu.sync_copy(rows[b], acc_sh.at[di_c[b]], add=True)

                @pl.when(j + b + nb < n_full)
                def _():
                    _unpack(pk_all, j + b + nb, si_c[b], di_c[b], cg)
                    pltpu.async_copy(h_hbm.at[si_c[b]], rows[b], g[b])

        if tail:
            @pl.loop(0, tail // LANES)
            def _(k):
                p = pk_all[pl.ds(n_full * cg + k * LANES, LANES)]
                si_t[pl.ds(k * LANES, LANES)] = jnp.bitwise_and(p, MASK16)
                di_t[pl.ds(k * LANES, LANES)] = jnp.right_shift(p, 16)
            pltpu.async_copy(h_hbm.at[si_t], rows_t, g[0]).wait()
            pltpu.sync_copy(rows_t, acc_sh.at[di_t], add=True)

        plsc.subcore_barrier()
        pltpu.sync_copy(acc_sh.at[pl.ds(s * rps, rps)],
                        out_hbm.at[c, pl.ds(s * rps, rps)])

    return agg_kernel


def _tc_mm_scale(x, W, dis, block=1000):
    n, k = x.shape
    d = W.shape[1]

    def body(x_ref, w_ref, dis_ref, o_ref):
        acc = jnp.dot(x_ref[...], w_ref[...], preferred_element_type=jnp.float32)
        o_ref[...] = acc * dis_ref[...]

    return pl.pallas_call(
        body,
        grid=(n // block,),
        in_specs=[
            pl.BlockSpec((block, k), lambda i: (i, 0)),
            pl.BlockSpec((k, d), lambda i: (0, 0)),
            pl.BlockSpec((block, 1), lambda i: (i, 0)),
        ],
        out_specs=pl.BlockSpec((block, d), lambda i: (i, 0)),
        out_shape=jax.ShapeDtypeStruct((n, d), jnp.float32),
    )(x, W, dis)


def _tc_combine_mm(acc, hp, dis, b, W, block=1000):
    n, k = hp.shape
    d = W.shape[1]

    def body(a0_ref, a1_ref, h_ref, dis_ref, b_ref, w_ref, o_ref):
        dis_c = dis_ref[...]
        z = (a0_ref[0] + a1_ref[0] + h_ref[...]) * dis_c + b_ref[...]
        z = jnp.maximum(z, 0.0)
        mm = jnp.dot(z, w_ref[...], preferred_element_type=jnp.float32)
        o_ref[...] = mm * dis_c

    return pl.pallas_call(
        body,
        grid=(n // block,),
        in_specs=[
            pl.BlockSpec((1, block, k), lambda i: (0, i, 0)),
            pl.BlockSpec((1, block, k), lambda i: (1, i, 0)),
            pl.BlockSpec((block, k), lambda i: (i, 0)),
            pl.BlockSpec((block, 1), lambda i: (i, 0)),
            pl.BlockSpec((1, k), lambda i: (0, 0)),
            pl.BlockSpec((k, d), lambda i: (0, 0)),
        ],
        out_specs=pl.BlockSpec((block, d), lambda i: (i, 0)),
        out_shape=jax.ShapeDtypeStruct((n, d), jnp.float32),
    )(acc, acc, hp, dis, b, W)


def _tc_final(acc, hp, dis, b, d_out, block=1000):
    n, d = hp.shape

    def body(a0_ref, a1_ref, h_ref, dis_ref, b_ref, o_ref):
        z = ((a0_ref[0] + a1_ref[0] + h_ref[...]) * dis_ref[...]
             + b_ref[...])
        o_ref[...] = z[:, :d_out]

    return pl.pallas_call(
        body,
        grid=(n // block,),
        in_specs=[
            pl.BlockSpec((1, block, d), lambda i: (0, i, 0)),
            pl.BlockSpec((1, block, d), lambda i: (1, i, 0)),
            pl.BlockSpec((block, d), lambda i: (i, 0)),
            pl.BlockSpec((block, 1), lambda i: (i, 0)),
            pl.BlockSpec((1, d), lambda i: (0, 0)),
        ],
        out_specs=pl.BlockSpec((block, d_out), lambda i: (i, 0)),
        out_shape=jax.ShapeDtypeStruct((n, d_out), jnp.float32),
    )(acc, acc, hp, dis, b)


def kernel(x, edge_index, W1, b1, W2, b2):
    n, _ = x.shape
    e = edge_index.shape[1]
    hid = W1.shape[1]
    out_d = W2.shape[1]
    n_pad = -(-n // (LANES * NS)) * (LANES * NS)

    src = edge_index[0]
    dst = edge_index[1]
    packed = jnp.bitwise_or(src, jnp.left_shift(dst, 16))
    zeros_hid = jnp.zeros((n_pad, hid), jnp.float32)
    zeros_out = jnp.zeros((n_pad, out_d), jnp.float32)

    degp = _make_sc_degree(e, n_pad)(packed)
    deg = degp[0, :n] + degp[1, :n] + 1.0
    dis = lax.rsqrt(deg)[:, None]

    h1p = _tc_mm_scale(x, W1, dis)
    acc1 = _make_sc_agg(e, n_pad, hid, cg=32, nb=8)(h1p, packed, zeros_hid)
    h2p = _tc_combine_mm(acc1, h1p, dis, b1.reshape(1, -1), W2)
    acc2 = _make_sc_agg(e, n_pad, out_d, tc_tiling=(out_d % 128 == 0),
                        cg=32, nb=8)(h2p, packed, zeros_out)
    return _tc_final(acc2, h2p, dis, b2.reshape(1, -1), out_d)


# speedup vs baseline: 1.0134x; 1.0134x over previous
import functools

import jax
import jax.numpy as jnp
from jax import lax
from jax.experimental import pallas as pl
from jax.experimental.pallas import tpu as pltpu
from jax.experimental.pallas import tpu_sc as plsc

NC = 2
NS = 16
NW = NC * NS
LANES = 16
CHUNK = 128
MASK16 = 0xFFFF


def _mesh():
    return plsc.VectorSubcoreMesh(core_axis_name="c", subcore_axis_name="s")


def _unpack(packed_all, j, si_c, di_c, cg=CHUNK):
    @pl.loop(0, cg // LANES)
    def _(k):
        p = packed_all[pl.ds(j * cg + k * LANES, LANES)]
        si_c[pl.ds(k * LANES, LANES)] = jnp.bitwise_and(p, MASK16)
        di_c[pl.ds(k * LANES, LANES)] = jnp.right_shift(p, 16)


def _make_sc_degree(E, N_pad):
    epw = E // NW
    n_full = epw // CHUNK
    tail = epw - n_full * CHUNK
    assert E % NW == 0 and epw % 8 == 0 and tail % LANES == 0
    rps = N_pad // NS
    assert rps % LANES == 0
    FIRE = 6
    assert n_full % FIRE == 0

    @functools.partial(
        pl.kernel,
        out_type=jax.ShapeDtypeStruct((NC, N_pad), jnp.float32),
        mesh=_mesh(),
        scratch_types=[
            pltpu.VMEM((epw,), jnp.int32),
            [pltpu.VMEM((CHUNK,), jnp.int32) for _ in range(FIRE)],
            pltpu.VMEM((CHUNK,), jnp.int32),
            pltpu.VMEM((CHUNK,), jnp.float32),
            pltpu.VMEM((rps,), jnp.float32),
            pltpu.VMEM_SHARED((N_pad,), jnp.float32),
            pltpu.SemaphoreType.DMA,
        ],
    )
    def deg_kernel(pk_hbm, out_hbm, pk_all, di_cs, di_scr, ones_v, zbuf_v,
                   acc_sh, sem):
        c = lax.axis_index("c")
        s = lax.axis_index("s")
        wid = c * NS + s

        @pl.loop(0, CHUNK // LANES)
        def _(i):
            ones_v[pl.ds(i * LANES, LANES)] = jnp.ones((LANES,), jnp.float32)

        @pl.loop(0, rps // LANES)
        def _(i):
            zbuf_v[pl.ds(i * LANES, LANES)] = jnp.zeros((LANES,), jnp.float32)

        pltpu.sync_copy(zbuf_v, acc_sh.at[pl.ds(s * rps, rps)])
        pltpu.sync_copy(pk_hbm.at[pl.ds(wid * epw, epw)], pk_all)
        plsc.subcore_barrier()

        @pl.loop(0, n_full // FIRE)
        def _(i):
            for b in range(FIRE):
                _unpack(pk_all, i * FIRE + b, di_scr, di_cs[b])
                pltpu.async_copy(ones_v, acc_sh.at[di_cs[b]], sem, add=True)
            for b in range(FIRE):
                pltpu.make_async_copy(ones_v, acc_sh.at[di_cs[0]], sem).wait()

        if tail:
            @pl.loop(0, tail // LANES)
            def _(k):
                p = pk_all[pl.ds(n_full * CHUNK + k * LANES, LANES)]
                di_cs[0][pl.ds(k * LANES, LANES)] = jnp.right_shift(p, 16)
            pltpu.sync_copy(ones_v.at[pl.ds(0, tail)],
                            acc_sh.at[di_cs[0].at[pl.ds(0, tail)]], add=True)

        plsc.subcore_barrier()
        pltpu.sync_copy(acc_sh.at[pl.ds(s * rps, rps)],
                        out_hbm.at[c, pl.ds(s * rps, rps)])

    return deg_kernel


def _make_sc_agg(E, N_pad, D, tc_tiling=True, cg=CHUNK, nb=2):
    epw = E // NW
    n_full = epw // cg
    tail = epw - n_full * cg
    assert E % NW == 0 and epw % 8 == 0 and tail % LANES == 0
    assert n_full % nb == 0
    rps = N_pad // NS
    t_sz = tail if tail else LANES

    @functools.partial(
        pl.kernel,
        out_type=jax.ShapeDtypeStruct((NC, N_pad, D), jnp.float32),
        mesh=_mesh(),
        scratch_types=[
            pltpu.VMEM((epw,), jnp.int32),
            [pltpu.VMEM((cg,), jnp.int32) for _ in range(nb)],
            [pltpu.VMEM((cg,), jnp.int32) for _ in range(nb)],
            [pltpu.VMEM((cg, D), jnp.float32) for _ in range(nb)],
            pltpu.VMEM((t_sz,), jnp.int32),
            pltpu.VMEM((t_sz,), jnp.int32),
            pltpu.VMEM((t_sz, D), jnp.float32),
            pltpu.VMEM_SHARED((N_pad, D), jnp.float32),
            [pltpu.SemaphoreType.DMA for _ in range(nb)],
        ],
        compiler_params=pltpu.CompilerParams(use_tc_tiling_on_sc=tc_tiling),
    )
    def agg_kernel(h_hbm, pk_hbm, zero_hbm, out_hbm,
                   pk_all, si_c, di_c, rows, si_t, di_t, rows_t, acc_sh, g):
        c = lax.axis_index("c")
        s = lax.axis_index("s")
        wid = c * NS + s

        pltpu.sync_copy(zero_hbm.at[pl.ds(s * rps, rps)],
                        acc_sh.at[pl.ds(s * rps, rps)])
        pltpu.sync_copy(pk_hbm.at[pl.ds(wid * epw, epw)], pk_all)
        plsc.subcore_barrier()

        for b in range(nb):
            _unpack(pk_all, b, si_c[b], di_c[b], cg)
            pltpu.async_copy(h_hbm.at[si_c[b]], rows[b], g[b])

        @pl.loop(0, n_full // nb)
        def _(i):
            j = i * nb
            for b in range(nb):
                pltpu.make_async_copy(h_hbm.at[si_c[b]], rows[b],
                                      g[b]).wait()
                pltpu.sync_copy(rows[b], acc_sh.at[di_c[b]], add=True)

                @pl.when(j + b + nb < n_full)
                def _():
                    _unpack(pk_all, j + b + nb, si_c[b], di_c[b], cg)
                    pltpu.async_copy(h_hbm.at[si_c[b]], rows[b], g[b])

        if tail:
            @pl.loop(0, tail // LANES)
            def _(k):
                p = pk_all[pl.ds(n_full * cg + k * LANES, LANES)]
                si_t[pl.ds(k * LANES, LANES)] = jnp.bitwise_and(p, MASK16)
                di_t[pl.ds(k * LANES, LANES)] = jnp.right_shift(p, 16)
            pltpu.async_copy(h_hbm.at[si_t], rows_t, g[0]).wait()
            pltpu.sync_copy(rows_t, acc_sh.at[di_t], add=True)

        plsc.subcore_barrier()
        pltpu.sync_copy(acc_sh.at[pl.ds(s * rps, rps)],
                        out_hbm.at[c, pl.ds(s * rps, rps)])

    return agg_kernel


def _tc_mm_scale(x, W, dis, block=1000):
    n, k = x.shape
    d = W.shape[1]

    def body(x_ref, w_ref, dis_ref, o_ref):
        acc = jnp.dot(x_ref[...], w_ref[...], preferred_element_type=jnp.float32)
        o_ref[...] = acc * dis_ref[...]

    return pl.pallas_call(
        body,
        grid=(n // block,),
        in_specs=[
            pl.BlockSpec((block, k), lambda i: (i, 0)),
            pl.BlockSpec((k, d), lambda i: (0, 0)),
            pl.BlockSpec((block, 1), lambda i: (i, 0)),
        ],
        out_specs=pl.BlockSpec((block, d), lambda i: (i, 0)),
        out_shape=jax.ShapeDtypeStruct((n, d), jnp.float32),
    )(x, W, dis)


def _tc_combine_mm(acc, hp, dis, b, W, block=1000):
    n, k = hp.shape
    d = W.shape[1]

    def body(a0_ref, a1_ref, h_ref, dis_ref, b_ref, w_ref, o_ref):
        dis_c = dis_ref[...]
        z = (a0_ref[0] + a1_ref[0] + h_ref[...]) * dis_c + b_ref[...]
        z = jnp.maximum(z, 0.0)
        mm = jnp.dot(z, w_ref[...], preferred_element_type=jnp.float32)
        o_ref[...] = mm * dis_c

    return pl.pallas_call(
        body,
        grid=(n // block,),
        in_specs=[
            pl.BlockSpec((1, block, k), lambda i: (0, i, 0)),
            pl.BlockSpec((1, block, k), lambda i: (1, i, 0)),
            pl.BlockSpec((block, k), lambda i: (i, 0)),
            pl.BlockSpec((block, 1), lambda i: (i, 0)),
            pl.BlockSpec((1, k), lambda i: (0, 0)),
            pl.BlockSpec((k, d), lambda i: (0, 0)),
        ],
        out_specs=pl.BlockSpec((block, d), lambda i: (i, 0)),
        out_shape=jax.ShapeDtypeStruct((n, d), jnp.float32),
    )(acc, acc, hp, dis, b, W)


def _tc_final(acc, hp, dis, b, d_out, block=1000):
    n, d = hp.shape

    def body(a0_ref, a1_ref, h_ref, dis_ref, b_ref, o_ref):
        z = ((a0_ref[0] + a1_ref[0] + h_ref[...]) * dis_ref[...]
             + b_ref[...])
        o_ref[...] = z[:, :d_out]

    return pl.pallas_call(
        body,
        grid=(n // block,),
        in_specs=[
            pl.BlockSpec((1, block, d), lambda i: (0, i, 0)),
            pl.BlockSpec((1, block, d), lambda i: (1, i, 0)),
            pl.BlockSpec((block, d), lambda i: (i, 0)),
            pl.BlockSpec((block, 1), lambda i: (i, 0)),
            pl.BlockSpec((1, d), lambda i: (0, 0)),
        ],
        out_specs=pl.BlockSpec((block, d_out), lambda i: (i, 0)),
        out_shape=jax.ShapeDtypeStruct((n, d_out), jnp.float32),
    )(acc, acc, hp, dis, b)


def kernel(x, edge_index, W1, b1, W2, b2):
    n, _ = x.shape
    e = edge_index.shape[1]
    hid = W1.shape[1]
    out_d = W2.shape[1]
    n_pad = -(-n // (LANES * NS)) * (LANES * NS)

    src = edge_index[0]
    dst = edge_index[1]
    packed = jnp.bitwise_or(src, jnp.left_shift(dst, 16))
    zeros_hid = jnp.zeros((n_pad, hid), jnp.float32)
    zeros_out = jnp.zeros((n_pad, out_d), jnp.float32)

    degp = _make_sc_degree(e, n_pad)(packed)
    deg = degp[0, :n] + degp[1, :n] + 1.0
    dis = lax.rsqrt(deg)[:, None]

    h1p = _tc_mm_scale(x, W1, dis)
    acc1 = _make_sc_agg(e, n_pad, hid, cg=64, nb=4)(h1p, packed, zeros_hid)
    h2p = _tc_combine_mm(acc1, h1p, dis, b1.reshape(1, -1), W2)
    acc2 = _make_sc_agg(e, n_pad, out_d, tc_tiling=(out_d % 128 == 0),
                        cg=64, nb=4)(h2p, packed, zeros_out)
    return _tc_final(acc2, h2p, dis, b2.reshape(1, -1), out_d)


# speedup vs baseline: 1.0874x; 1.0730x over previous
import functools

import jax
import jax.numpy as jnp
from jax import lax
from jax.experimental import pallas as pl
from jax.experimental.pallas import tpu as pltpu
from jax.experimental.pallas import tpu_sc as plsc

NC = 2
NS = 16
NW = NC * NS
LANES = 16
CHUNK = 128
MASK16 = 0xFFFF


def _mesh():
    return plsc.VectorSubcoreMesh(core_axis_name="c", subcore_axis_name="s")


def _unpack(packed_all, j, si_c, di_c, cg=CHUNK):
    @pl.loop(0, cg // LANES)
    def _(k):
        p = packed_all[pl.ds(j * cg + k * LANES, LANES)]
        si_c[pl.ds(k * LANES, LANES)] = jnp.bitwise_and(p, MASK16)
        di_c[pl.ds(k * LANES, LANES)] = jnp.right_shift(p, 16)


def _make_sc_degree(E, N_pad):
    epw = E // NW
    n_full = epw // CHUNK
    tail = epw - n_full * CHUNK
    assert E % NW == 0 and epw % 8 == 0 and tail % LANES == 0
    rps = N_pad // NS
    assert rps % LANES == 0
    FIRE = 6
    assert n_full % FIRE == 0

    @functools.partial(
        pl.kernel,
        out_type=jax.ShapeDtypeStruct((NC, N_pad), jnp.float32),
        mesh=_mesh(),
        scratch_types=[
            pltpu.VMEM((epw,), jnp.int32),
            [pltpu.VMEM((CHUNK,), jnp.int32) for _ in range(FIRE)],
            pltpu.VMEM((CHUNK,), jnp.int32),
            pltpu.VMEM((CHUNK,), jnp.float32),
            pltpu.VMEM((rps,), jnp.float32),
            pltpu.VMEM_SHARED((N_pad,), jnp.float32),
            pltpu.SemaphoreType.DMA,
        ],
    )
    def deg_kernel(pk_hbm, out_hbm, pk_all, di_cs, di_scr, ones_v, zbuf_v,
                   acc_sh, sem):
        c = lax.axis_index("c")
        s = lax.axis_index("s")
        wid = c * NS + s

        @pl.loop(0, CHUNK // LANES)
        def _(i):
            ones_v[pl.ds(i * LANES, LANES)] = jnp.ones((LANES,), jnp.float32)

        @pl.loop(0, rps // LANES)
        def _(i):
            zbuf_v[pl.ds(i * LANES, LANES)] = jnp.zeros((LANES,), jnp.float32)

        pltpu.sync_copy(zbuf_v, acc_sh.at[pl.ds(s * rps, rps)])
        pltpu.sync_copy(pk_hbm.at[pl.ds(wid * epw, epw)], pk_all)
        plsc.subcore_barrier()

        @pl.loop(0, n_full // FIRE)
        def _(i):
            for b in range(FIRE):
                _unpack(pk_all, i * FIRE + b, di_scr, di_cs[b])
                pltpu.async_copy(ones_v, acc_sh.at[di_cs[b]], sem, add=True)
            for b in range(FIRE):
                pltpu.make_async_copy(ones_v, acc_sh.at[di_cs[0]], sem).wait()

        if tail:
            @pl.loop(0, tail // LANES)
            def _(k):
                p = pk_all[pl.ds(n_full * CHUNK + k * LANES, LANES)]
                di_cs[0][pl.ds(k * LANES, LANES)] = jnp.right_shift(p, 16)
            pltpu.sync_copy(ones_v.at[pl.ds(0, tail)],
                            acc_sh.at[di_cs[0].at[pl.ds(0, tail)]], add=True)

        plsc.subcore_barrier()
        pltpu.sync_copy(acc_sh.at[pl.ds(s * rps, rps)],
                        out_hbm.at[c, pl.ds(s * rps, rps)])

    return deg_kernel


def _make_sc_agg(E, N_pad, D, tc_tiling=True, cg=CHUNK, nb=2):
    epw = E // NW
    n_full = epw // cg
    tail = epw - n_full * cg
    assert E % NW == 0 and epw % 8 == 0 and tail % LANES == 0
    assert n_full % nb == 0
    rps = N_pad // NS
    assert rps % cg == 0
    t_sz = tail if tail else LANES

    @functools.partial(
        pl.kernel,
        out_type=jax.ShapeDtypeStruct((NC, N_pad, D), jnp.float32),
        mesh=_mesh(),
        scratch_types=[
            pltpu.VMEM((epw,), jnp.int32),
            [pltpu.VMEM((cg,), jnp.int32) for _ in range(nb)],
            [pltpu.VMEM((cg,), jnp.int32) for _ in range(nb)],
            [pltpu.VMEM((cg, D), jnp.float32) for _ in range(nb)],
            pltpu.VMEM((t_sz,), jnp.int32),
            pltpu.VMEM((t_sz,), jnp.int32),
            pltpu.VMEM((t_sz, D), jnp.float32),
            pltpu.VMEM_SHARED((N_pad, D), jnp.float32),
            [pltpu.SemaphoreType.DMA for _ in range(nb)],
        ],
        compiler_params=pltpu.CompilerParams(use_tc_tiling_on_sc=tc_tiling),
    )
    def agg_kernel(h_hbm, pk_hbm, out_hbm,
                   pk_all, si_c, di_c, rows, si_t, di_t, rows_t, acc_sh, g):
        c = lax.axis_index("c")
        s = lax.axis_index("s")
        wid = c * NS + s

        pltpu.async_copy(pk_hbm.at[pl.ds(wid * epw, epw)], pk_all, g[0])

        @pl.loop(0, cg)
        def _(i):
            for jj in range(D // LANES):
                rows[0].at[pl.ds(i, 1), pl.ds(jj * LANES, LANES)][...] = (
                    jnp.zeros((1, LANES), jnp.float32))

        @pl.loop(0, rps // cg)
        def _(i):
            pltpu.sync_copy(rows[0], acc_sh.at[pl.ds(s * rps + i * cg, cg)])

        pltpu.make_async_copy(pk_hbm.at[pl.ds(wid * epw, epw)], pk_all,
                              g[0]).wait()
        plsc.subcore_barrier()

        for b in range(nb):
            _unpack(pk_all, b, si_c[b], di_c[b], cg)
            pltpu.async_copy(h_hbm.at[si_c[b]], rows[b], g[b])

        @pl.loop(0, n_full // nb)
        def _(i):
            j = i * nb
            for b in range(nb):
                pltpu.make_async_copy(h_hbm.at[si_c[b]], rows[b],
                                      g[b]).wait()
                pltpu.sync_copy(rows[b], acc_sh.at[di_c[b]], add=True)

                @pl.when(j + b + nb < n_full)
                def _():
                    _unpack(pk_all, j + b + nb, si_c[b], di_c[b], cg)
                    pltpu.async_copy(h_hbm.at[si_c[b]], rows[b], g[b])

        if tail:
            @pl.loop(0, tail // LANES)
            def _(k):
                p = pk_all[pl.ds(n_full * cg + k * LANES, LANES)]
                si_t[pl.ds(k * LANES, LANES)] = jnp.bitwise_and(p, MASK16)
                di_t[pl.ds(k * LANES, LANES)] = jnp.right_shift(p, 16)
            pltpu.async_copy(h_hbm.at[si_t], rows_t, g[0]).wait()
            pltpu.sync_copy(rows_t, acc_sh.at[di_t], add=True)

        plsc.subcore_barrier()
        pltpu.sync_copy(acc_sh.at[pl.ds(s * rps, rps)],
                        out_hbm.at[c, pl.ds(s * rps, rps)])

    return agg_kernel


def _tc_mm_scale(x, W, dis, block=2000):
    n, k = x.shape
    d = W.shape[1]

    def body(x_ref, w_ref, dis_ref, o_ref):
        acc = jnp.dot(x_ref[...], w_ref[...], preferred_element_type=jnp.float32)
        o_ref[...] = acc * dis_ref[...]

    return pl.pallas_call(
        body,
        grid=(n // block,),
        in_specs=[
            pl.BlockSpec((block, k), lambda i: (i, 0)),
            pl.BlockSpec((k, d), lambda i: (0, 0)),
            pl.BlockSpec((block, 1), lambda i: (i, 0)),
        ],
        out_specs=pl.BlockSpec((block, d), lambda i: (i, 0)),
        out_shape=jax.ShapeDtypeStruct((n, d), jnp.float32),
    )(x, W, dis)


def _tc_combine_mm(acc, hp, dis, b, W, block=2000):
    n, k = hp.shape
    d = W.shape[1]

    def body(a0_ref, a1_ref, h_ref, dis_ref, b_ref, w_ref, o_ref):
        dis_c = dis_ref[...]
        z = (a0_ref[0] + a1_ref[0] + h_ref[...]) * dis_c + b_ref[...]
        z = jnp.maximum(z, 0.0)
        mm = jnp.dot(z, w_ref[...], preferred_element_type=jnp.float32)
        o_ref[...] = mm * dis_c

    return pl.pallas_call(
        body,
        grid=(n // block,),
        in_specs=[
            pl.BlockSpec((1, block, k), lambda i: (0, i, 0)),
            pl.BlockSpec((1, block, k), lambda i: (1, i, 0)),
            pl.BlockSpec((block, k), lambda i: (i, 0)),
            pl.BlockSpec((block, 1), lambda i: (i, 0)),
            pl.BlockSpec((1, k), lambda i: (0, 0)),
            pl.BlockSpec((k, d), lambda i: (0, 0)),
        ],
        out_specs=pl.BlockSpec((block, d), lambda i: (i, 0)),
        out_shape=jax.ShapeDtypeStruct((n, d), jnp.float32),
    )(acc, acc, hp, dis, b, W)


def _tc_final(acc, hp, dis, b, d_out, block=2000):
    n, d = hp.shape

    def body(a0_ref, a1_ref, h_ref, dis_ref, b_ref, o_ref):
        z = ((a0_ref[0] + a1_ref[0] + h_ref[...]) * dis_ref[...]
             + b_ref[...])
        o_ref[...] = z[:, :d_out]

    return pl.pallas_call(
        body,
        grid=(n // block,),
        in_specs=[
            pl.BlockSpec((1, block, d), lambda i: (0, i, 0)),
            pl.BlockSpec((1, block, d), lambda i: (1, i, 0)),
            pl.BlockSpec((block, d), lambda i: (i, 0)),
            pl.BlockSpec((block, 1), lambda i: (i, 0)),
            pl.BlockSpec((1, d), lambda i: (0, 0)),
        ],
        out_specs=pl.BlockSpec((block, d_out), lambda i: (i, 0)),
        out_shape=jax.ShapeDtypeStruct((n, d_out), jnp.float32),
    )(acc, acc, hp, dis, b)


def kernel(x, edge_index, W1, b1, W2, b2):
    n, _ = x.shape
    e = edge_index.shape[1]
    hid = W1.shape[1]
    out_d = W2.shape[1]
    n_pad = -(-n // (LANES * NS)) * (LANES * NS)

    src = edge_index[0]
    dst = edge_index[1]
    packed = jnp.bitwise_or(src, jnp.left_shift(dst, 16))

    degp = _make_sc_degree(e, n_pad)(packed)
    deg = degp[0, :n] + degp[1, :n] + 1.0
    dis = lax.rsqrt(deg)[:, None]

    h1p = _tc_mm_scale(x, W1, dis)
    acc1 = _make_sc_agg(e, n_pad, hid, cg=64, nb=4)(h1p, packed)
    h2p = _tc_combine_mm(acc1, h1p, dis, b1.reshape(1, -1), W2)
    acc2 = _make_sc_agg(e, n_pad, out_d, tc_tiling=(out_d % 128 == 0),
                        cg=64, nb=4)(h2p, packed)
    return _tc_final(acc2, h2p, dis, b2.reshape(1, -1), out_d)


# speedup vs baseline: 1.1020x; 1.0134x over previous
import functools

import jax
import jax.numpy as jnp
from jax import lax
from jax.experimental import pallas as pl
from jax.experimental.pallas import tpu as pltpu
from jax.experimental.pallas import tpu_sc as plsc

NC = 2
NS = 16
NW = NC * NS
LANES = 16
CHUNK = 128
MASK16 = 0xFFFF


def _mesh():
    return plsc.VectorSubcoreMesh(core_axis_name="c", subcore_axis_name="s")


def _unpack(packed_all, j, si_c, di_c, cg=CHUNK):
    @pl.loop(0, cg // LANES)
    def _(k):
        p = packed_all[pl.ds(j * cg + k * LANES, LANES)]
        si_c[pl.ds(k * LANES, LANES)] = jnp.bitwise_and(p, MASK16)
        di_c[pl.ds(k * LANES, LANES)] = jnp.right_shift(p, 16)


def _make_sc_degree(E, N_pad):
    epw = E // NW
    n_full = epw // CHUNK
    tail = epw - n_full * CHUNK
    assert E % NW == 0 and epw % 8 == 0 and tail % LANES == 0
    rps = N_pad // NS
    assert rps % LANES == 0
    FIRE = 6
    assert n_full % FIRE == 0

    @functools.partial(
        pl.kernel,
        out_type=jax.ShapeDtypeStruct((NC, N_pad), jnp.float32),
        mesh=_mesh(),
        scratch_types=[
            pltpu.VMEM((epw,), jnp.int32),
            [pltpu.VMEM((CHUNK,), jnp.int32) for _ in range(FIRE)],
            pltpu.VMEM((CHUNK,), jnp.int32),
            pltpu.VMEM((CHUNK,), jnp.float32),
            pltpu.VMEM((rps,), jnp.float32),
            pltpu.VMEM_SHARED((N_pad,), jnp.float32),
            pltpu.SemaphoreType.DMA,
        ],
    )
    def deg_kernel(pk_hbm, out_hbm, pk_all, di_cs, di_scr, ones_v, zbuf_v,
                   acc_sh, sem):
        c = lax.axis_index("c")
        s = lax.axis_index("s")
        wid = c * NS + s

        @pl.loop(0, CHUNK // LANES)
        def _(i):
            ones_v[pl.ds(i * LANES, LANES)] = jnp.ones((LANES,), jnp.float32)

        @pl.loop(0, rps // LANES)
        def _(i):
            zbuf_v[pl.ds(i * LANES, LANES)] = jnp.zeros((LANES,), jnp.float32)

        pltpu.sync_copy(zbuf_v, acc_sh.at[pl.ds(s * rps, rps)])
        pltpu.sync_copy(pk_hbm.at[pl.ds(wid * epw, epw)], pk_all)
        plsc.subcore_barrier()

        @pl.loop(0, n_full // FIRE)
        def _(i):
            for b in range(FIRE):
                _unpack(pk_all, i * FIRE + b, di_scr, di_cs[b])
                pltpu.async_copy(ones_v, acc_sh.at[di_cs[b]], sem, add=True)
            for b in range(FIRE):
                pltpu.make_async_copy(ones_v, acc_sh.at[di_cs[0]], sem).wait()

        if tail:
            @pl.loop(0, tail // LANES)
            def _(k):
                p = pk_all[pl.ds(n_full * CHUNK + k * LANES, LANES)]
                di_cs[0][pl.ds(k * LANES, LANES)] = jnp.right_shift(p, 16)
            pltpu.sync_copy(ones_v.at[pl.ds(0, tail)],
                            acc_sh.at[di_cs[0].at[pl.ds(0, tail)]], add=True)

        plsc.subcore_barrier()
        pltpu.sync_copy(acc_sh.at[pl.ds(s * rps, rps)],
                        out_hbm.at[c, pl.ds(s * rps, rps)])

    return deg_kernel


def _make_sc_agg(E, N_pad, D, tc_tiling=True, cg=CHUNK, nb=2):
    epw = E // NW
    n_full = epw // cg
    tail = epw - n_full * cg
    assert E % NW == 0 and epw % 8 == 0 and tail % LANES == 0
    assert n_full % nb == 0
    rps = N_pad // NS
    assert rps % cg == 0
    t_sz = tail if tail else LANES

    @functools.partial(
        pl.kernel,
        out_type=jax.ShapeDtypeStruct((NC, N_pad, D), jnp.float32),
        mesh=_mesh(),
        scratch_types=[
            pltpu.VMEM((epw,), jnp.int32),
            [pltpu.VMEM((cg,), jnp.int32) for _ in range(nb)],
            [pltpu.VMEM((cg,), jnp.int32) for _ in range(nb)],
            [pltpu.VMEM((cg, D), jnp.float32) for _ in range(nb)],
            pltpu.VMEM((t_sz,), jnp.int32),
            pltpu.VMEM((t_sz,), jnp.int32),
            pltpu.VMEM((t_sz, D), jnp.float32),
            pltpu.VMEM_SHARED((N_pad, D), jnp.float32),
            [pltpu.SemaphoreType.DMA for _ in range(nb)],
        ],
        compiler_params=pltpu.CompilerParams(use_tc_tiling_on_sc=tc_tiling),
    )
    def agg_kernel(h_hbm, pk_hbm, out_hbm,
                   pk_all, si_c, di_c, rows, si_t, di_t, rows_t, acc_sh, g):
        c = lax.axis_index("c")
        s = lax.axis_index("s")
        wid = c * NS + s

        pltpu.async_copy(pk_hbm.at[pl.ds(wid * epw, epw)], pk_all, g[0])

        @pl.loop(0, cg)
        def _(i):
            for jj in range(D // LANES):
                rows[0].at[pl.ds(i, 1), pl.ds(jj * LANES, LANES)][...] = (
                    jnp.zeros((1, LANES), jnp.float32))

        @pl.loop(0, rps // cg)
        def _(i):
            pltpu.sync_copy(rows[0], acc_sh.at[pl.ds(s * rps + i * cg, cg)])

        pltpu.make_async_copy(pk_hbm.at[pl.ds(wid * epw, epw)], pk_all,
                              g[0]).wait()
        plsc.subcore_barrier()

        for b in range(nb):
            _unpack(pk_all, b, si_c[b], di_c[b], cg)
            pltpu.async_copy(h_hbm.at[si_c[b]], rows[b], g[b])

        @pl.loop(0, n_full // nb)
        def _(i):
            j = i * nb
            for b in range(nb):
                pltpu.make_async_copy(h_hbm.at[si_c[b]], rows[b],
                                      g[b]).wait()
                pltpu.sync_copy(rows[b], acc_sh.at[di_c[b]], add=True)

                @pl.when(j + b + nb < n_full)
                def _():
                    _unpack(pk_all, j + b + nb, si_c[b], di_c[b], cg)
                    pltpu.async_copy(h_hbm.at[si_c[b]], rows[b], g[b])

        if tail:
            @pl.loop(0, tail // LANES)
            def _(k):
                p = pk_all[pl.ds(n_full * cg + k * LANES, LANES)]
                si_t[pl.ds(k * LANES, LANES)] = jnp.bitwise_and(p, MASK16)
                di_t[pl.ds(k * LANES, LANES)] = jnp.right_shift(p, 16)
            pltpu.async_copy(h_hbm.at[si_t], rows_t, g[0]).wait()
            pltpu.sync_copy(rows_t, acc_sh.at[di_t], add=True)

        plsc.subcore_barrier()
        pltpu.sync_copy(acc_sh.at[pl.ds(s * rps, rps)],
                        out_hbm.at[c, pl.ds(s * rps, rps)])

    return agg_kernel


def _tc_mm(x, W, block=2000):
    n, k = x.shape
    d = W.shape[1]

    def body(x_ref, w_ref, o_ref):
        o_ref[...] = jnp.dot(x_ref[...], w_ref[...],
                             preferred_element_type=jnp.float32)

    return pl.pallas_call(
        body,
        grid=(n // block,),
        in_specs=[
            pl.BlockSpec((block, k), lambda i: (i, 0)),
            pl.BlockSpec((k, d), lambda i: (0, 0)),
        ],
        out_specs=pl.BlockSpec((block, d), lambda i: (i, 0)),
        out_shape=jax.ShapeDtypeStruct((n, d), jnp.float32),
    )(x, W)


def _tc_combine_mm(acc, hp, dis, b, W, block=2000):
    n, k = hp.shape
    d = W.shape[1]

    def body(a0_ref, a1_ref, h_ref, dis_ref, b_ref, w_ref, o_ref):
        dis_c = dis_ref[...]
        z = (a0_ref[0] + a1_ref[0] + h_ref[...]) * dis_c + b_ref[...]
        z = jnp.maximum(z, 0.0)
        mm = jnp.dot(z, w_ref[...], preferred_element_type=jnp.float32)
        o_ref[...] = mm * dis_c

    return pl.pallas_call(
        body,
        grid=(n // block,),
        in_specs=[
            pl.BlockSpec((1, block, k), lambda i: (0, i, 0)),
            pl.BlockSpec((1, block, k), lambda i: (1, i, 0)),
            pl.BlockSpec((block, k), lambda i: (i, 0)),
            pl.BlockSpec((block, 1), lambda i: (i, 0)),
            pl.BlockSpec((1, k), lambda i: (0, 0)),
            pl.BlockSpec((k, d), lambda i: (0, 0)),
        ],
        out_specs=pl.BlockSpec((block, d), lambda i: (i, 0)),
        out_shape=jax.ShapeDtypeStruct((n, d), jnp.float32),
    )(acc, acc, hp, dis, b, W)


def _tc_final(acc, hp, dis, b, d_out, block=2000):
    n, d = hp.shape

    def body(a0_ref, a1_ref, h_ref, dis_ref, b_ref, o_ref):
        z = ((a0_ref[0] + a1_ref[0] + h_ref[...]) * dis_ref[...]
             + b_ref[...])
        o_ref[...] = z[:, :d_out]

    return pl.pallas_call(
        body,
        grid=(n // block,),
        in_specs=[
            pl.BlockSpec((1, block, d), lambda i: (0, i, 0)),
            pl.BlockSpec((1, block, d), lambda i: (1, i, 0)),
            pl.BlockSpec((block, d), lambda i: (i, 0)),
            pl.BlockSpec((block, 1), lambda i: (i, 0)),
            pl.BlockSpec((1, d), lambda i: (0, 0)),
        ],
        out_specs=pl.BlockSpec((block, d_out), lambda i: (i, 0)),
        out_shape=jax.ShapeDtypeStruct((n, d_out), jnp.float32),
    )(acc, acc, hp, dis, b)


def kernel(x, edge_index, W1, b1, W2, b2):
    n, _ = x.shape
    e = edge_index.shape[1]
    hid = W1.shape[1]
    out_d = W2.shape[1]
    n_pad = -(-n // (LANES * NS)) * (LANES * NS)

    src = edge_index[0]
    dst = edge_index[1]
    packed = jnp.bitwise_or(src, jnp.left_shift(dst, 16))

    degp = _make_sc_degree(e, n_pad)(packed)
    deg = degp[0, :n] + degp[1, :n] + 1.0
    dis = lax.rsqrt(deg)[:, None]

    h1p = dis * _tc_mm(x, W1)
    acc1 = _make_sc_agg(e, n_pad, hid, cg=64, nb=4)(h1p, packed)
    h2p = _tc_combine_mm(acc1, h1p, dis, b1.reshape(1, -1), W2)
    acc2 = _make_sc_agg(e, n_pad, out_d, tc_tiling=(out_d % 128 == 0),
                        cg=64, nb=4)(h2p, packed)
    return _tc_final(acc2, h2p, dis, b2.reshape(1, -1), out_d)
